# baseline (speedup 1.0000x reference)
import functools

import jax
import jax.numpy as jnp
from jax import lax
from jax.experimental import pallas as pl
from jax.experimental.pallas import tpu as pltpu
from jax.experimental.pallas import tpu_sc as plsc

B = 16384
D = 128
NC = 2
NS = 16
NW = NC * NS
BPW = B // NW
CH = 64
NCH = BPW // CH
GRP = 16


def _tt_body(uid_hbm, mid_hbm, ut_hbm, mt_hbm, out_hbm,
             uidx_v, midx_v, urows_v, mrows_v, col_v, out_v, sem, isem):
    wid = lax.axis_index("s") * NC + lax.axis_index("c")
    base = wid * BPW
    ci = pltpu.async_copy(uid_hbm.at[pl.ds(base, BPW)], uidx_v, isem)
    cm = pltpu.async_copy(mid_hbm.at[pl.ds(base, BPW)], midx_v, isem)
    ci.wait()
    cm.wait()

    def fire(j):
        b = j % 2
        return (
            pltpu.async_copy(
                ut_hbm.at[uidx_v.at[pl.ds(j * CH, CH)]], urows_v.at[b], sem),
            pltpu.async_copy(
                mt_hbm.at[midx_v.at[pl.ds(j * CH, CH)]], mrows_v.at[b], sem),
        )

    pend = fire(0)
    for j in range(NCH):
        for c in pend:
            c.wait()
        if j + 1 < NCH:
            pend = fire(j + 1)
        b = j % 2

        def group(g, carry, j=j, b=b):
            lanes = lax.iota(jnp.int32, GRP)
            cbase = (g % 2) * (GRP * GRP)
            for r in range(GRP):
                row = g * GRP + r
                acc = (urows_v[b, row, pl.ds(0, 16)]
                       * mrows_v[b, row, pl.ds(0, 16)])
                for q in range(1, D // 16):
                    acc = acc + (urows_v[b, row, pl.ds(q * 16, 16)]
                                 * mrows_v[b, row, pl.ds(q * 16, 16)])
                plsc.store_scatter(col_v, [cbase + lanes * GRP + r], acc)
            vec = col_v[pl.ds(cbase, GRP)]
            for l in range(1, GRP):
                vec = vec + col_v[pl.ds(cbase + l * GRP, GRP)]
            out_v[pl.ds(j * CH + g * GRP, GRP)] = 1.0 / (1.0 + jnp.exp(-vec))
            return carry

        lax.fori_loop(0, CH // GRP, group, 0, unroll=2)
    pltpu.sync_copy(out_v, out_hbm.at[pl.ds(base, BPW)])


_two_tower = functools.partial(
    pl.kernel,
    out_type=jax.ShapeDtypeStruct((B,), jnp.float32),
    mesh=plsc.VectorSubcoreMesh(core_axis_name="c", subcore_axis_name="s"),
    compiler_params=pltpu.CompilerParams(
        needs_layout_passes=False,
        disable_bounds_checks=True,
        disable_semaphore_checks=True,
    ),
    scratch_types=[
        pltpu.VMEM((BPW,), jnp.int32),
        pltpu.VMEM((BPW,), jnp.int32),
        pltpu.VMEM((2, CH, D), jnp.float32),
        pltpu.VMEM((2, CH, D), jnp.float32),
        pltpu.VMEM((2 * GRP * GRP,), jnp.float32),
        pltpu.VMEM((BPW,), jnp.float32),
        pltpu.SemaphoreType.DMA,
        pltpu.SemaphoreType.DMA,
    ],
)(_tt_body)


def kernel(user_ids, movie_ids, genre_vectors, user_table, movie_table,
           genre_W, genre_b):
    del genre_vectors, genre_W, genre_b
    return _two_tower(user_ids.astype(jnp.int32), movie_ids.astype(jnp.int32),
                      user_table, movie_table)


# speedup vs baseline: 1.3013x; 1.3013x over previous
import functools

import jax
import jax.numpy as jnp
from jax import lax
from jax.experimental import pallas as pl
from jax.experimental.pallas import tpu as pltpu
from jax.experimental.pallas import tpu_sc as plsc

B = 16384
D = 128
NC = 2
NS = 16
NW = NC * NS
BPW = B // NW
CH = 128
NCH = BPW // CH
GRP = 16


def _tt_body(uid_hbm, mid_hbm, ut_hbm, mt_hbm, out_hbm,
             uidx_v, midx_v, urows_v, mrows_v, col_v, out_v, sem, isem):
    wid = lax.axis_index("s") * NC + lax.axis_index("c")
    base = wid * BPW
    ci = pltpu.async_copy(uid_hbm.at[pl.ds(base, BPW)], uidx_v, isem)
    cm = pltpu.async_copy(mid_hbm.at[pl.ds(base, BPW)], midx_v, isem)
    ci.wait()
    cm.wait()

    def fire(j):
        b = j % 2
        return (
            pltpu.async_copy(
                ut_hbm.at[uidx_v.at[pl.ds(j * CH, CH)]], urows_v.at[b], sem),
            pltpu.async_copy(
                mt_hbm.at[midx_v.at[pl.ds(j * CH, CH)]], mrows_v.at[b], sem),
        )

    pend = fire(0)
    for j in range(NCH):
        for c in pend:
            c.wait()
        if j + 1 < NCH:
            pend = fire(j + 1)
        b = j % 2

        def group(g, carry, j=j, b=b):
            lanes = lax.iota(jnp.int32, GRP)
            for r in range(GRP):
                row = g * GRP + r
                acc = (urows_v[b, row, pl.ds(0, 16)]
                       * mrows_v[b, row, pl.ds(0, 16)])
                for q in range(1, D // 16):
                    acc = acc + (urows_v[b, row, pl.ds(q * 16, 16)]
                                 * mrows_v[b, row, pl.ds(q * 16, 16)])
                plsc.store_scatter(col_v, [lanes * GRP + r], acc)
            vec = col_v[pl.ds(0, GRP)]
            for l in range(1, GRP):
                vec = vec + col_v[pl.ds(l * GRP, GRP)]
            out_v[pl.ds(j * CH + g * GRP, GRP)] = 1.0 / (1.0 + jnp.exp(-vec))
            return carry

        lax.fori_loop(0, CH // GRP, group, 0)
    pltpu.sync_copy(out_v, out_hbm.at[pl.ds(base, BPW)])


_two_tower = functools.partial(
    pl.kernel,
    out_type=jax.ShapeDtypeStruct((B,), jnp.float32),
    mesh=plsc.VectorSubcoreMesh(core_axis_name="c", subcore_axis_name="s"),
    compiler_params=pltpu.CompilerParams(
        needs_layout_passes=False,
        disable_bounds_checks=True,
        disable_semaphore_checks=True,
    ),
    scratch_types=[
        pltpu.VMEM((BPW,), jnp.int32),
        pltpu.VMEM((BPW,), jnp.int32),
        pltpu.VMEM((2, CH, D), jnp.float32),
        pltpu.VMEM((2, CH, D), jnp.float32),
        pltpu.VMEM((GRP * GRP,), jnp.float32),
        pltpu.VMEM((BPW,), jnp.float32),
        pltpu.SemaphoreType.DMA,
        pltpu.SemaphoreType.DMA,
    ],
)(_tt_body)


def kernel(user_ids, movie_ids, genre_vectors, user_table, movie_table,
           genre_W, genre_b):
    del genre_vectors, genre_W, genre_b
    return _two_tower(user_ids.astype(jnp.int32), movie_ids.astype(jnp.int32),
                      user_table, movie_table)


# speedup vs baseline: 1.3024x; 1.0009x over previous
import functools

import jax
import jax.numpy as jnp
from jax import lax
from jax.experimental import pallas as pl
from jax.experimental.pallas import tpu as pltpu
from jax.experimental.pallas import tpu_sc as plsc

B = 16384
D = 128
NC = 2
NS = 16
NW = NC * NS
BPW = B // NW
CH = 128
NCH = BPW // CH
GRP = 16


def _tt_body(uid_hbm, mid_hbm, ut_hbm, mt_hbm, out_hbm,
             uidx_v, midx_v, urows_v, mrows_v, col_v, out_v, sem, isem):
    wid = lax.axis_index("s") * NC + lax.axis_index("c")
    base = wid * BPW
    ci = pltpu.async_copy(uid_hbm.at[pl.ds(base, BPW)], uidx_v, isem)
    cm = pltpu.async_copy(mid_hbm.at[pl.ds(base, BPW)], midx_v, isem)
    ci.wait()
    cm.wait()

    def fire(j):
        b = j % 2
        return (
            pltpu.async_copy(
                ut_hbm.at[uidx_v.at[pl.ds(j * CH, CH)]], urows_v.at[b], sem),
            pltpu.async_copy(
                mt_hbm.at[midx_v.at[pl.ds(j * CH, CH)]], mrows_v.at[b], sem),
        )

    pend = fire(0)
    for j in range(NCH):
        for c in pend:
            c.wait()
        if j + 1 < NCH:
            pend = fire(j + 1)
        b = j % 2

        def group(g, carry, j=j, b=b):
            lanes = lax.iota(jnp.int32, GRP)
            for r in range(GRP):
                row = g * GRP + r
                acc = (urows_v[b, row, pl.ds(0, 16)]
                       * mrows_v[b, row, pl.ds(0, 16)])
                for q in range(1, D // 16):
                    acc = acc + (urows_v[b, row, pl.ds(q * 16, 16)]
                                 * mrows_v[b, row, pl.ds(q * 16, 16)])
                plsc.store_scatter(col_v, [lanes * GRP + r], acc)
            vec = col_v[pl.ds(0, GRP)]
            for l in range(1, GRP):
                vec = vec + col_v[pl.ds(l * GRP, GRP)]
            out_v[pl.ds(j * CH + g * GRP, GRP)] = 1.0 / (1.0 + jnp.exp(-vec))
            return carry

        lax.fori_loop(0, CH // GRP, group, 0)
    pltpu.sync_copy(out_v, out_hbm.at[pl.ds(base, BPW)])


_two_tower = functools.partial(
    pl.kernel,
    out_type=jax.ShapeDtypeStruct((B,), jnp.float32),
    mesh=plsc.VectorSubcoreMesh(core_axis_name="c", subcore_axis_name="s"),
    compiler_params=pltpu.CompilerParams(
        needs_layout_passes=False,
        disable_bounds_checks=True,
        disable_semaphore_checks=True,
        use_tc_tiling_on_sc=False,
    ),
    scratch_types=[
        pltpu.VMEM((BPW,), jnp.int32),
        pltpu.VMEM((BPW,), jnp.int32),
        pltpu.VMEM((2, CH, D), jnp.float32),
        pltpu.VMEM((2, CH, D), jnp.float32),
        pltpu.VMEM((GRP * GRP,), jnp.float32),
        pltpu.VMEM((BPW,), jnp.float32),
        pltpu.SemaphoreType.DMA,
        pltpu.SemaphoreType.DMA,
    ],
)(_tt_body)


def kernel(user_ids, movie_ids, genre_vectors, user_table, movie_table,
           genre_W, genre_b):
    del genre_vectors, genre_W, genre_b
    return _two_tower(user_ids.astype(jnp.int32), movie_ids.astype(jnp.int32),
                      user_table, movie_table)


# speedup vs baseline: 1.3054x; 1.0023x over previous
import functools

import jax
import jax.numpy as jnp
from jax import lax
from jax.experimental import pallas as pl
from jax.experimental.pallas import tpu as pltpu
from jax.experimental.pallas import tpu_sc as plsc

B = 16384
D = 128
NC = 2
NS = 16
NW = NC * NS
BPW = B // NW
CH = 128
NCH = BPW // CH
GRP = 16


def _tt_body(uid_hbm, mid_hbm, ut_hbm, mt_hbm, out_hbm,
             uidx_v, midx_v, urows_v, mrows_v, col_v, out_v, sem, isem):
    wid = lax.axis_index("s") * NC + lax.axis_index("c")
    base = wid * BPW
    ci = pltpu.async_copy(uid_hbm.at[pl.ds(base, BPW)], uidx_v, isem)
    cm = pltpu.async_copy(mid_hbm.at[pl.ds(base, BPW)], midx_v, isem)
    ci.wait()
    cm.wait()

    NBUF = 3
    AHEAD = 2

    def fire(j):
        b = j % NBUF
        return (
            pltpu.async_copy(
                ut_hbm.at[uidx_v.at[pl.ds(j * CH, CH)]], urows_v.at[b], sem),
            pltpu.async_copy(
                mt_hbm.at[midx_v.at[pl.ds(j * CH, CH)]], mrows_v.at[b], sem),
        )

    pend = {j: fire(j) for j in range(AHEAD)}
    for j in range(NCH):
        for c in pend.pop(j):
            c.wait()
        if j + AHEAD < NCH:
            pend[j + AHEAD] = fire(j + AHEAD)
        b = j % NBUF

        def group(g, carry, j=j, b=b):
            lanes = lax.iota(jnp.int32, GRP)
            for r in range(GRP):
                row = g * GRP + r
                acc = (urows_v[b, row, pl.ds(0, 16)]
                       * mrows_v[b, row, pl.ds(0, 16)])
                for q in range(1, D // 16):
                    acc = acc + (urows_v[b, row, pl.ds(q * 16, 16)]
                                 * mrows_v[b, row, pl.ds(q * 16, 16)])
                plsc.store_scatter(col_v, [lanes * GRP + r], acc)
            vec = col_v[pl.ds(0, GRP)]
            for l in range(1, GRP):
                vec = vec + col_v[pl.ds(l * GRP, GRP)]
            out_v[pl.ds(j * CH + g * GRP, GRP)] = 1.0 / (1.0 + jnp.exp(-vec))
            return carry

        lax.fori_loop(0, CH // GRP, group, 0)
    pltpu.sync_copy(out_v, out_hbm.at[pl.ds(base, BPW)])


_two_tower = functools.partial(
    pl.kernel,
    out_type=jax.ShapeDtypeStruct((B,), jnp.float32),
    mesh=plsc.VectorSubcoreMesh(core_axis_name="c", subcore_axis_name="s"),
    compiler_params=pltpu.CompilerParams(
        needs_layout_passes=False,
        disable_bounds_checks=True,
        disable_semaphore_checks=True,
    ),
    scratch_types=[
        pltpu.VMEM((BPW,), jnp.int32),
        pltpu.VMEM((BPW,), jnp.int32),
        pltpu.VMEM((3, CH, D), jnp.float32),
        pltpu.VMEM((3, CH, D), jnp.float32),
        pltpu.VMEM((GRP * GRP,), jnp.float32),
        pltpu.VMEM((BPW,), jnp.float32),
        pltpu.SemaphoreType.DMA,
        pltpu.SemaphoreType.DMA,
    ],
)(_tt_body)


def kernel(user_ids, movie_ids, genre_vectors, user_table, movie_table,
           genre_W, genre_b):
    del genre_vectors, genre_W, genre_b
    return _two_tower(user_ids.astype(jnp.int32), movie_ids.astype(jnp.int32),
                      user_table, movie_table)


# speedup vs baseline: 1.3849x; 1.0609x over previous
import functools

import jax
import jax.numpy as jnp
from jax import lax
from jax.experimental import pallas as pl
from jax.experimental.pallas import tpu as pltpu
from jax.experimental.pallas import tpu_sc as plsc

B = 16384
D = 128
NC = 2
NS = 16
NW = NC * NS
BPW = B // NW
CH = 128
NCH = BPW // CH
GRP = 16


def _tt_body(uid_hbm, mid_hbm, ut_hbm, mt_hbm, out_hbm,
             uidx_v, midx_v, urows_v, mrows_v, col_v, out_v, sem, isem):
    wid = lax.axis_index("s") * NC + lax.axis_index("c")
    base = wid * BPW
    ci = pltpu.async_copy(uid_hbm.at[pl.ds(base, BPW)], uidx_v, isem)
    cm = pltpu.async_copy(mid_hbm.at[pl.ds(base, BPW)], midx_v, isem)
    ci.wait()
    cm.wait()

    def fire(j, b):
        pltpu.async_copy(
            ut_hbm.at[uidx_v.at[pl.ds(j * CH, CH)]], urows_v.at[b], sem)
        pltpu.async_copy(
            mt_hbm.at[midx_v.at[pl.ds(j * CH, CH)]], mrows_v.at[b], sem)

    def drain(b):
        pltpu.make_async_copy(
            ut_hbm.at[uidx_v.at[pl.ds(0, CH)]], urows_v.at[b], sem).wait()
        pltpu.make_async_copy(
            mt_hbm.at[midx_v.at[pl.ds(0, CH)]], mrows_v.at[b], sem).wait()

    fire(0, 0)
    fire(1, 1)

    def outer(g, carry):
        for b in range(2):
            j = 2 * g + b
            drain(b)

            @pl.when(j + 2 < NCH)
            def _(j=j, b=b):
                fire(j + 2, b)

            def group(gg, carry2, b=b):
                lanes = lax.iota(jnp.int32, GRP)
                for r in range(GRP):
                    row = gg * GRP + r
                    acc = (urows_v[b, row, pl.ds(0, 16)]
                           * mrows_v[b, row, pl.ds(0, 16)])
                    for q in range(1, D // 16):
                        acc = acc + (urows_v[b, row, pl.ds(q * 16, 16)]
                                     * mrows_v[b, row, pl.ds(q * 16, 16)])
                    plsc.store_scatter(col_v, [lanes * GRP + r], acc)
                vec = col_v[pl.ds(0, GRP)]
                for l in range(1, GRP):
                    vec = vec + col_v[pl.ds(l * GRP, GRP)]
                out_v[pl.ds(j * CH + gg * GRP, GRP)] = (
                    1.0 / (1.0 + jnp.exp(-vec)))
                return carry2

            lax.fori_loop(0, CH // GRP, group, 0)
        return carry

    lax.fori_loop(0, NCH // 2, outer, 0)
    pltpu.sync_copy(out_v, out_hbm.at[pl.ds(base, BPW)])


_two_tower = functools.partial(
    pl.kernel,
    out_type=jax.ShapeDtypeStruct((B,), jnp.float32),
    mesh=plsc.VectorSubcoreMesh(core_axis_name="c", subcore_axis_name="s"),
    compiler_params=pltpu.CompilerParams(
        needs_layout_passes=False,
        disable_bounds_checks=True,
        disable_semaphore_checks=True,
    ),
    scratch_types=[
        pltpu.VMEM((BPW,), jnp.int32),
        pltpu.VMEM((BPW,), jnp.int32),
        pltpu.VMEM((2, CH, D), jnp.float32),
        pltpu.VMEM((2, CH, D), jnp.float32),
        pltpu.VMEM((GRP * GRP,), jnp.float32),
        pltpu.VMEM((BPW,), jnp.float32),
        pltpu.SemaphoreType.DMA,
        pltpu.SemaphoreType.DMA,
    ],
)(_tt_body)


def kernel(user_ids, movie_ids, genre_vectors, user_table, movie_table,
           genre_W, genre_b):
    del genre_vectors, genre_W, genre_b
    return _two_tower(user_ids.astype(jnp.int32), movie_ids.astype(jnp.int32),
                      user_table, movie_table)
